# baseline (speedup 1.0000x reference)
import functools

import jax
import jax.numpy as jnp
from jax import lax
from jax.experimental import pallas as pl
from jax.experimental.pallas import tpu as pltpu
from jax.experimental.pallas import tpu_sc as plsc

LANES = 16
BBLK = 128
NRING = 4
LOOK = 3


@functools.lru_cache(maxsize=None)
def _make_embed(num_pos: int, batch: int, hidden: int):
    info = plsc.get_sparse_core_info()
    nc, ns = info.num_cores, info.num_subcores
    nw = nc * ns
    assert num_pos % nw == 0
    assert batch % BBLK == 0
    assert hidden % 8 == 0 and hidden % LANES == 0
    hp = hidden
    ppw = num_pos // nw
    kb = batch // BBLK
    assert kb == 8
    sub_rows = hidden // 8

    mesh = plsc.VectorSubcoreMesh(core_axis_name="c", subcore_axis_name="s")

    @functools.partial(
        pl.kernel,
        out_type=jax.ShapeDtypeStruct((num_pos, sub_rows, kb, 8, BBLK), jnp.float32),
        mesh=mesh,
        scratch_types=[
            pltpu.VMEM((2, batch), jnp.int32),
            pltpu.VMEM((NRING, BBLK, hp), jnp.float32),
            pltpu.VMEM((NRING, sub_rows, 8, BBLK), jnp.float32),
            pltpu.VMEM((ppw, hidden), jnp.float32),
            pltpu.SemaphoreType.DMA((2,)),
            pltpu.SemaphoreType.DMA((NRING,)),
            pltpu.SemaphoreType.DMA((NRING,)),
        ],
        compiler_params=pltpu.CompilerParams(
            use_tc_tiling_on_sc=False, needs_layout_passes=False),
    )
    def embed(idx_hbm, table_hbm, pos_hbm, out_hbm,
              idx_db, g_ring, o_ring, pos_v, idx_sem, g_sem, st_sem):
        wid = lax.axis_index("s") * nc + lax.axis_index("c")
        hw0 = wid * ppw

        pltpu.sync_copy(pos_hbm.at[pl.ds(hw0, ppw)], pos_v)

        iota = lax.iota(jnp.int32, LANES)
        row_vecs = [iota + LANES * g for g in range(BBLK // LANES)]

        def stage_idx(p, slot):
            pltpu.make_async_copy(
                idx_hbm.at[hw0 + p], idx_db.at[slot], idx_sem.at[slot]).start()

        def gather(p_slot, k, gslot):
            return pltpu.make_async_copy(
                table_hbm.at[idx_db.at[p_slot, pl.ds(BBLK * k, BBLK)]],
                g_ring.at[gslot], g_sem.at[gslot])

        def store(p, k, s):
            return pltpu.make_async_copy(
                o_ring.at[s], out_hbm.at[hw0 + p, :, k], st_sem.at[s])

        stage_idx(0, 0)
        stage_idx(1, 1)
        pltpu.make_async_copy(
            idx_hbm.at[hw0], idx_db.at[0], idx_sem.at[0]).wait()
        for k in range(LOOK):
            gather(0, k, k).start()

        @pl.loop(0, ppw)
        def ploop(p):
            pm2 = lax.rem(p, 2)
            pm2n = 1 - pm2
            psplat = jnp.broadcast_to(p, (LANES,)).astype(jnp.int32)

            @pl.when(p + 1 < ppw)
            def _():
                pltpu.make_async_copy(
                    idx_hbm.at[hw0], idx_db.at[pm2n], idx_sem.at[pm2n]).wait()

            for bb in range(kb):
                rs = bb % NRING
                ahead = bb + LOOK
                gslot = ahead % NRING

                if ahead < kb:
                    gather(pm2, ahead, gslot).start()
                else:

                    @pl.when(p + 1 < ppw)
                    def _():
                        gather(pm2n, ahead - kb, gslot).start()

                gather(pm2, bb, rs).wait()

                if bb >= NRING:
                    store(p, bb, rs).wait()
                else:

                    @pl.when(p > 0)
                    def _():
                        store(p, bb, rs).wait()

                @pl.loop(0, sub_rows)
                def trloop(tr):
                    for sub in range(8):
                        d = tr * 8 + sub
                        cols = jnp.broadcast_to(d, (LANES,)).astype(jnp.int32)
                        pv = plsc.load_gather(pos_v, [psplat, cols])
                        for g in range(BBLK // LANES):
                            vals = plsc.load_gather(
                                g_ring.at[rs], [row_vecs[g], cols])
                            o_ring[rs, tr, sub, pl.ds(LANES * g, LANES)] = vals + pv

                store(p, bb, rs).start()

            @pl.when(p + 2 < ppw)
            def _():
                stage_idx(p + 2, pm2)

        for s in range(NRING):
            store(0, 0, s).wait()

    return embed


def kernel(input_grid, tok_table, pos_embed):
    b, h, w = input_grid.shape
    hidden = tok_table.shape[1]
    hw = h * w
    idx_t = input_grid.reshape(b, hw).T
    pos_flat = pos_embed[0, :h, :w, :].reshape(hw, hidden)
    embed = _make_embed(hw, b, hidden)
    raw = embed(idx_t, tok_table, pos_flat)
    return (
        raw.reshape(h, w, hidden // 8, b // BBLK, 8, BBLK)
        .transpose(3, 5, 0, 1, 2, 4)
        .reshape(b, h, w, hidden)
    )


# speedup vs baseline: 1.4414x; 1.4414x over previous
import functools

import jax
import jax.numpy as jnp
from jax import lax
from jax.experimental import pallas as pl
from jax.experimental.pallas import tpu as pltpu
from jax.experimental.pallas import tpu_sc as plsc

LANES = 16
CHUNK = 128
NRING = 7
GL = 5
NIDX = 8
IL = 7


@functools.lru_cache(maxsize=None)
def _make_embed(nrows: int, num_pos: int, hidden: int):
    info = plsc.get_sparse_core_info()
    nc, ns = info.num_cores, info.num_subcores
    nw = nc * ns
    assert nrows % (nw * CHUNK) == 0
    rpw = nrows // nw
    assert num_pos % CHUNK == 0 and rpw % num_pos == 0
    nchunks = rpw // CHUNK
    pc = num_pos // CHUNK
    assert hidden % LANES == 0
    assert nchunks > IL and IL > GL and IL < NIDX and GL < NRING

    mesh = plsc.VectorSubcoreMesh(core_axis_name="c", subcore_axis_name="s")

    @functools.partial(
        pl.kernel,
        out_type=jax.ShapeDtypeStruct((nrows, hidden), jnp.float32),
        mesh=mesh,
        scratch_types=[
            pltpu.VMEM((NIDX, CHUNK), jnp.int32),
            pltpu.VMEM((NRING, CHUNK, hidden), jnp.float32),
            pltpu.VMEM((num_pos, hidden), jnp.float32),
            pltpu.SemaphoreType.DMA((NIDX,)),
            pltpu.SemaphoreType.DMA((NRING,)),
            pltpu.SemaphoreType.DMA((NRING,)),
        ],
        compiler_params=pltpu.CompilerParams(
            use_tc_tiling_on_sc=False, needs_layout_passes=False),
    )
    def embed(idx_hbm, table_hbm, pos_hbm, out_hbm,
              idx_ring, g_ring, pos_v, idx_sem, g_sem, st_sem):
        wid = lax.axis_index("s") * nc + lax.axis_index("c")
        row0 = wid * rpw

        pltpu.sync_copy(pos_hbm, pos_v)

        def stage(j, s):
            return pltpu.make_async_copy(
                idx_hbm.at[pl.ds(row0 + CHUNK * j, CHUNK)],
                idx_ring.at[s], idx_sem.at[s])

        def gather(s):
            return pltpu.make_async_copy(
                table_hbm.at[idx_ring.at[s % NIDX]],
                g_ring.at[s % NRING], g_sem.at[s % NRING])

        def store(j, s):
            return pltpu.make_async_copy(
                g_ring.at[s], out_hbm.at[pl.ds(row0 + CHUNK * j, CHUNK)],
                st_sem.at[s])

        for j in range(IL):
            stage(j, j).start()
        for j in range(GL):
            stage(j, j).wait()
            gather(j).start()

        @pl.loop(0, nchunks)
        def cloop(c):
            @pl.when(c + IL < nchunks)
            def _():
                stage(c + IL, lax.rem(c + IL, NIDX)).start()

            j = c + GL

            @pl.when(j < nchunks)
            def _():
                stage(j, lax.rem(j, NIDX)).wait()

                @pl.when(j >= NRING)
                def _():
                    store(0, lax.rem(j, NRING)).wait()

                gather(j).start()

            s = lax.rem(c, NRING)
            gather(c).wait()

            po = CHUNK * lax.rem(c, pc)

            @pl.loop(0, CHUNK)
            def rloop(r):
                for dv in range(hidden // LANES):
                    col = pl.ds(LANES * dv, LANES)
                    plsc.addupdate(g_ring.at[s, r, col], pos_v[po + r, col])

            store(c, s).start()

        for s in range(NRING):
            store(0, s).wait()

    return embed


def kernel(input_grid, tok_table, pos_embed):
    b, h, w = input_grid.shape
    hidden = tok_table.shape[1]
    hw = h * w
    idx_flat = input_grid.reshape(b * hw)
    pos_flat = pos_embed[0, :h, :w, :].reshape(hw, hidden)
    embed = _make_embed(b * hw, hw, hidden)
    raw = embed(idx_flat, tok_table, pos_flat)
    return raw.reshape(b, h, w, hidden)


# speedup vs baseline: 2.5587x; 1.7752x over previous
import functools

import jax
import jax.numpy as jnp
from jax import lax
from jax.experimental import pallas as pl
from jax.experimental.pallas import tpu as pltpu
from jax.experimental.pallas import tpu_sc as plsc

LANES = 16
BBLK = 128
NRING = 4
LOOK = 3


@functools.lru_cache(maxsize=None)
def _make_embed(num_pos: int, batch: int, hidden: int):
    info = plsc.get_sparse_core_info()
    nc, ns = info.num_cores, info.num_subcores
    nw = nc * ns
    assert num_pos % nw == 0
    assert batch % BBLK == 0
    assert hidden % LANES == 0
    ppw = num_pos // nw
    kb = batch // BBLK
    assert kb == 8
    sub_rows = hidden // 8
    ndv = hidden // LANES

    mesh = plsc.VectorSubcoreMesh(core_axis_name="c", subcore_axis_name="s")

    @functools.partial(
        pl.kernel,
        out_type=jax.ShapeDtypeStruct((num_pos, sub_rows, kb, 8, BBLK), jnp.float32),
        mesh=mesh,
        scratch_types=[
            pltpu.VMEM((2, batch), jnp.int32),
            pltpu.VMEM((NRING, BBLK, hidden), jnp.float32),
            pltpu.VMEM((NRING, sub_rows, 8, BBLK), jnp.float32),
            pltpu.VMEM((ppw, hidden), jnp.float32),
            pltpu.SemaphoreType.DMA((2,)),
            pltpu.SemaphoreType.DMA((NRING,)),
            pltpu.SemaphoreType.DMA((NRING,)),
        ],
        compiler_params=pltpu.CompilerParams(
            use_tc_tiling_on_sc=False, needs_layout_passes=False),
    )
    def embed(idx_hbm, table_hbm, pos_hbm, out_hbm,
              idx_db, g_ring, o_ring, pos_v, idx_sem, g_sem, st_sem):
        wid = lax.axis_index("s") * nc + lax.axis_index("c")
        hw0 = wid * ppw

        pltpu.sync_copy(pos_hbm.at[pl.ds(hw0, ppw)], pos_v)

        lane = lax.iota(jnp.int32, LANES)
        rperm = [lax.rem(lane + j, LANES) for j in range(LANES)]
        dcol = [lane + LANES * dv for dv in range(ndv)]
        trow = [lax.div(lane, 8) + (LANES // 8) * dv for dv in range(ndv)]
        sub = lax.rem(lane, 8)

        def stage_idx(p, slot):
            pltpu.make_async_copy(
                idx_hbm.at[hw0 + p], idx_db.at[slot], idx_sem.at[slot]).start()

        def gather(p_slot, k, gslot):
            return pltpu.make_async_copy(
                table_hbm.at[idx_db.at[p_slot, pl.ds(BBLK * k, BBLK)]],
                g_ring.at[gslot], g_sem.at[gslot])

        def store(p, k, s):
            return pltpu.make_async_copy(
                o_ring.at[s], out_hbm.at[hw0 + p, :, k], st_sem.at[s])

        stage_idx(0, 0)
        stage_idx(1, 1)
        pltpu.make_async_copy(
            idx_hbm.at[hw0], idx_db.at[0], idx_sem.at[0]).wait()
        for k in range(LOOK):
            gather(0, k, k).start()

        @pl.loop(0, ppw)
        def ploop(p):
            pm2 = lax.rem(p, 2)
            pm2n = 1 - pm2
            psplat = jnp.broadcast_to(p, (LANES,)).astype(jnp.int32)
            pv = [plsc.load_gather(pos_v, [psplat, dcol[dv]])
                  for dv in range(ndv)]

            @pl.when(p + 1 < ppw)
            def _():
                pltpu.make_async_copy(
                    idx_hbm.at[hw0], idx_db.at[pm2n], idx_sem.at[pm2n]).wait()

            for bb in range(kb):
                rs = bb % NRING
                ahead = bb + LOOK
                gslot = ahead % NRING

                if ahead < kb:
                    gather(pm2, ahead, gslot).start()
                else:

                    @pl.when(p + 1 < ppw)
                    def _():
                        gather(pm2n, ahead - kb, gslot).start()

                gather(pm2, bb, rs).wait()

                if bb >= NRING:
                    store(p, bb, rs).wait()
                else:

                    @pl.when(p > 0)
                    def _():
                        store(p, bb, rs).wait()

                @pl.loop(0, BBLK // LANES)
                def bloop(bg):
                    bgs = jnp.broadcast_to(bg * LANES, (LANES,)).astype(
                        jnp.int32)
                    for j in range(LANES):
                        rows = rperm[j] + bgs
                        for dv in range(ndv):
                            vals = plsc.load_gather(
                                g_ring.at[rs], [rows, dcol[dv]])
                            plsc.store_scatter(
                                o_ring.at[rs], [trow[dv], sub, rows],
                                vals + pv[dv])

                store(p, bb, rs).start()

            @pl.when(p + 2 < ppw)
            def _():
                stage_idx(p + 2, pm2)

        for s in range(NRING):
            store(0, 0, s).wait()

    return embed


def kernel(input_grid, tok_table, pos_embed):
    b, h, w = input_grid.shape
    hidden = tok_table.shape[1]
    hw = h * w
    idx_t = input_grid.reshape(b, hw).T
    pos_flat = pos_embed[0, :h, :w, :].reshape(hw, hidden)
    embed = _make_embed(hw, b, hidden)
    raw = embed(idx_t, tok_table, pos_flat)
    return (
        raw.reshape(h, w, hidden // 8, b // BBLK, 8, BBLK)
        .transpose(3, 5, 0, 1, 2, 4)
        .reshape(b, h, w, hidden)
    )
